# baseline (speedup 1.0000x reference)
import functools

import jax
import jax.numpy as jnp
from jax import lax
from jax.experimental import pallas as pl
from jax.experimental.pallas import tpu as pltpu
from jax.experimental.pallas import tpu_sc as plsc

ROWS = 65536
COLS = 256
LANES = 16
GROUPS = COLS // LANES
NUM_CORES = 2
NUM_SUBCORES = 16
NUM_WORKERS = NUM_CORES * NUM_SUBCORES
ROWS_PER_WORKER = ROWS // NUM_WORKERS
CHUNK = 64
NUM_CHUNKS = ROWS_PER_WORKER // CHUNK
NBUF = 2
NUM_PAIRS = NUM_CHUNKS // NBUF


def _permute_chunk(in_b, out_b, idx_groups):
    @plsc.parallel_loop(0, CHUNK, unroll=8)
    def row_body(r):
        row_v = jnp.full((LANES,), r, jnp.int32)
        for g in range(GROUPS):
            vals = plsc.load_gather(in_b, [row_v, idx_groups[g]])
            out_b[r, pl.ds(g * LANES, LANES)] = vals


def _permute_body(x_hbm, perm_hbm, y_hbm, perm_v, in_v, out_v, sem_in, sem_out):
    wid = lax.axis_index("s") * NUM_CORES + lax.axis_index("c")
    base_row = wid * ROWS_PER_WORKER
    pltpu.sync_copy(perm_hbm, perm_v)

    idx_groups = [perm_v[pl.ds(g * LANES, LANES)] for g in range(GROUPS)]

    def in_slice(ci):
        return x_hbm.at[pl.ds(base_row + ci * CHUNK, CHUNK)]

    def out_slice(ci):
        return y_hbm.at[pl.ds(base_row + ci * CHUNK, CHUNK)]

    def in_start(ci, b):
        pltpu.async_copy(in_slice(ci), in_v[b], sem_in[b])

    def in_wait(ci, b):
        pltpu.make_async_copy(in_slice(ci), in_v[b], sem_in[b]).wait()

    def out_start(ci, b):
        pltpu.async_copy(out_v[b], out_slice(ci), sem_out[b])

    def out_wait(ci, b):
        pltpu.make_async_copy(out_v[b], out_slice(ci), sem_out[b]).wait()

    for b in range(NBUF):
        in_start(b, b)

    for b in range(NBUF):
        in_wait(b, b)
        _permute_chunk(in_v[b], out_v[b], idx_groups)
        out_start(b, b)
        in_start(b + NBUF, b)

    def pair_body(p, carry):
        for b in range(NBUF):
            ci = p * NBUF + b
            in_wait(ci, b)
            out_wait(ci - NBUF, b)
            _permute_chunk(in_v[b], out_v[b], idx_groups)
            out_start(ci, b)
            in_start(ci + NBUF, b)
        return carry

    lax.fori_loop(1, NUM_PAIRS - 1, pair_body, 0)

    for b in range(NBUF):
        ci = NUM_CHUNKS - NBUF + b
        in_wait(ci, b)
        out_wait(ci - NBUF, b)
        _permute_chunk(in_v[b], out_v[b], idx_groups)
        out_start(ci, b)

    for b in range(NBUF):
        out_wait(NUM_CHUNKS - NBUF + b, b)


_permute_call = functools.partial(
    pl.kernel,
    out_type=jax.ShapeDtypeStruct((ROWS, COLS), jnp.float32),
    mesh=plsc.VectorSubcoreMesh(
        core_axis_name="c",
        subcore_axis_name="s",
        num_cores=NUM_CORES,
        num_subcores=NUM_SUBCORES,
    ),
    scratch_types=[
        pltpu.VMEM((COLS,), jnp.int32),
        [pltpu.VMEM((CHUNK, COLS), jnp.float32) for _ in range(NBUF)],
        [pltpu.VMEM((CHUNK, COLS), jnp.float32) for _ in range(NBUF)],
        [pltpu.SemaphoreType.DMA for _ in range(NBUF)],
        [pltpu.SemaphoreType.DMA for _ in range(NBUF)],
    ],
    compiler_params=pltpu.CompilerParams(needs_layout_passes=False),
)(_permute_body)


def kernel(x, perm):
    y = _permute_call(x, perm.astype(jnp.int32))
    logdet = jnp.zeros(x.shape[0], dtype=x.dtype)
    return (y, logdet)


# speedup vs baseline: 1.2204x; 1.2204x over previous
import functools

import jax
import jax.numpy as jnp
from jax import lax
from jax.experimental import pallas as pl
from jax.experimental.pallas import tpu as pltpu
from jax.experimental.pallas import tpu_sc as plsc

ROWS = 65536
COLS = 256
LANES = 16
GROUPS = COLS // LANES
NUM_CORES = 2
NUM_SUBCORES = 16
NUM_WORKERS = NUM_CORES * NUM_SUBCORES
ROWS_PER_WORKER = ROWS // NUM_WORKERS
CHUNK = 64
NUM_CHUNKS = ROWS_PER_WORKER // CHUNK
NBUF = 2
NUM_PAIRS = NUM_CHUNKS // NBUF


def _permute_chunk(in_b, out_b, idx_groups):
    @plsc.parallel_loop(0, CHUNK, unroll=2)
    def row_body(r):
        row_v = jnp.full((LANES,), r, jnp.int32)
        for g in range(GROUPS):
            vals = plsc.load_gather(in_b, [row_v, idx_groups[g]])
            out_b[r, pl.ds(g * LANES, LANES)] = vals


def _permute_body(x_hbm, perm_hbm, y_hbm, perm_v, in_v, out_v, sem_in, sem_out):
    wid = lax.axis_index("s") * NUM_CORES + lax.axis_index("c")
    base_row = wid * ROWS_PER_WORKER
    pltpu.sync_copy(perm_hbm, perm_v)

    idx_groups = [perm_v[pl.ds(g * LANES, LANES)] for g in range(GROUPS)]

    def in_slice(ci):
        return x_hbm.at[pl.ds(base_row + ci * CHUNK, CHUNK)]

    def out_slice(ci):
        return y_hbm.at[pl.ds(base_row + ci * CHUNK, CHUNK)]

    def in_start(ci, b):
        pltpu.async_copy(in_slice(ci), in_v[b], sem_in[b])

    def in_wait(ci, b):
        pltpu.make_async_copy(in_slice(ci), in_v[b], sem_in[b]).wait()

    def out_start(ci, b):
        pltpu.async_copy(out_v[b], out_slice(ci), sem_out[b])

    def out_wait(ci, b):
        pltpu.make_async_copy(out_v[b], out_slice(ci), sem_out[b]).wait()

    for b in range(NBUF):
        in_start(b, b)

    for b in range(NBUF):
        in_wait(b, b)
        _permute_chunk(in_v[b], out_v[b], idx_groups)
        out_start(b, b)
        in_start(b + NBUF, b)

    def pair_body(p, carry):
        for b in range(NBUF):
            ci = p * NBUF + b
            in_wait(ci, b)
            out_wait(ci - NBUF, b)
            _permute_chunk(in_v[b], out_v[b], idx_groups)
            out_start(ci, b)
            in_start(ci + NBUF, b)
        return carry

    lax.fori_loop(1, NUM_PAIRS - 1, pair_body, 0)

    for b in range(NBUF):
        ci = NUM_CHUNKS - NBUF + b
        in_wait(ci, b)
        out_wait(ci - NBUF, b)
        _permute_chunk(in_v[b], out_v[b], idx_groups)
        out_start(ci, b)

    for b in range(NBUF):
        out_wait(NUM_CHUNKS - NBUF + b, b)


_permute_call = functools.partial(
    pl.kernel,
    out_type=jax.ShapeDtypeStruct((ROWS, COLS), jnp.float32),
    mesh=plsc.VectorSubcoreMesh(
        core_axis_name="c",
        subcore_axis_name="s",
        num_cores=NUM_CORES,
        num_subcores=NUM_SUBCORES,
    ),
    scratch_types=[
        pltpu.VMEM((COLS,), jnp.int32),
        [pltpu.VMEM((CHUNK, COLS), jnp.float32) for _ in range(NBUF)],
        [pltpu.VMEM((CHUNK, COLS), jnp.float32) for _ in range(NBUF)],
        [pltpu.SemaphoreType.DMA for _ in range(NBUF)],
        [pltpu.SemaphoreType.DMA for _ in range(NBUF)],
    ],
    compiler_params=pltpu.CompilerParams(needs_layout_passes=False),
)(_permute_body)


def kernel(x, perm):
    y = _permute_call(x, perm.astype(jnp.int32))
    logdet = jnp.zeros(x.shape[0], dtype=x.dtype)
    return (y, logdet)


# speedup vs baseline: 1.2312x; 1.0089x over previous
import functools

import jax
import jax.numpy as jnp
from jax import lax
from jax.experimental import pallas as pl
from jax.experimental.pallas import tpu as pltpu
from jax.experimental.pallas import tpu_sc as plsc

ROWS = 65536
COLS = 256
LANES = 16
GROUPS = COLS // LANES
NUM_CORES = 2
NUM_SUBCORES = 16
NUM_WORKERS = NUM_CORES * NUM_SUBCORES
ROWS_PER_WORKER = ROWS // NUM_WORKERS
CHUNK = 64
NUM_CHUNKS = ROWS_PER_WORKER // CHUNK
NBUF = 2
NUM_PAIRS = NUM_CHUNKS // NBUF


def _permute_chunk(in_b, out_b, idx_groups):
    @plsc.parallel_loop(0, CHUNK)
    def row_body(r):
        row_v = jnp.full((LANES,), r, jnp.int32)
        for g in range(GROUPS):
            vals = plsc.load_gather(in_b, [row_v, idx_groups[g]])
            out_b[r, pl.ds(g * LANES, LANES)] = vals


def _permute_body(x_hbm, perm_hbm, y_hbm, perm_v, in_v, out_v, sem_in, sem_out):
    wid = lax.axis_index("s") * NUM_CORES + lax.axis_index("c")
    base_row = wid * ROWS_PER_WORKER
    pltpu.sync_copy(perm_hbm, perm_v)

    idx_groups = [perm_v[pl.ds(g * LANES, LANES)] for g in range(GROUPS)]

    def in_slice(ci):
        return x_hbm.at[pl.ds(base_row + ci * CHUNK, CHUNK)]

    def out_slice(ci):
        return y_hbm.at[pl.ds(base_row + ci * CHUNK, CHUNK)]

    def in_start(ci, b):
        pltpu.async_copy(in_slice(ci), in_v[b], sem_in[b])

    def in_wait(ci, b):
        pltpu.make_async_copy(in_slice(ci), in_v[b], sem_in[b]).wait()

    def out_start(ci, b):
        pltpu.async_copy(out_v[b], out_slice(ci), sem_out[b])

    def out_wait(ci, b):
        pltpu.make_async_copy(out_v[b], out_slice(ci), sem_out[b]).wait()

    for b in range(NBUF):
        in_start(b, b)

    for b in range(NBUF):
        in_wait(b, b)
        _permute_chunk(in_v[b], out_v[b], idx_groups)
        out_start(b, b)
        in_start(b + NBUF, b)

    def pair_body(p, carry):
        for b in range(NBUF):
            ci = p * NBUF + b
            in_wait(ci, b)
            out_wait(ci - NBUF, b)
            _permute_chunk(in_v[b], out_v[b], idx_groups)
            out_start(ci, b)
            in_start(ci + NBUF, b)
        return carry

    lax.fori_loop(1, NUM_PAIRS - 1, pair_body, 0)

    for b in range(NBUF):
        ci = NUM_CHUNKS - NBUF + b
        in_wait(ci, b)
        out_wait(ci - NBUF, b)
        _permute_chunk(in_v[b], out_v[b], idx_groups)
        out_start(ci, b)

    for b in range(NBUF):
        out_wait(NUM_CHUNKS - NBUF + b, b)


_permute_call = functools.partial(
    pl.kernel,
    out_type=jax.ShapeDtypeStruct((ROWS, COLS), jnp.float32),
    mesh=plsc.VectorSubcoreMesh(
        core_axis_name="c",
        subcore_axis_name="s",
        num_cores=NUM_CORES,
        num_subcores=NUM_SUBCORES,
    ),
    scratch_types=[
        pltpu.VMEM((COLS,), jnp.int32),
        [pltpu.VMEM((CHUNK, COLS), jnp.float32) for _ in range(NBUF)],
        [pltpu.VMEM((CHUNK, COLS), jnp.float32) for _ in range(NBUF)],
        [pltpu.SemaphoreType.DMA for _ in range(NBUF)],
        [pltpu.SemaphoreType.DMA for _ in range(NBUF)],
    ],
    compiler_params=pltpu.CompilerParams(needs_layout_passes=False),
)(_permute_body)


def kernel(x, perm):
    y = _permute_call(x, perm.astype(jnp.int32))
    logdet = jnp.zeros(x.shape[0], dtype=x.dtype)
    return (y, logdet)
